# baseline (speedup 1.0000x reference)
import jax
import jax.numpy as jnp
from jax.experimental import pallas as pl

_ROWS, _COLS, _D = 64, 64, 32
_N = _ROWS * _COLS
_BB = 1024


def _bmu_block(x_ref, w_ref, out_ref):
    xb = x_ref[...]
    wt = w_ref[...].T
    wn2 = jnp.sum(wt * wt, axis=0, keepdims=True)
    dots = jnp.dot(
        xb, wt,
        preferred_element_type=jnp.float32,
        precision=jax.lax.Precision.HIGHEST,
    )
    s = wn2 - 2.0 * dots
    m = jnp.min(s, axis=1, keepdims=True)
    ii = jax.lax.broadcasted_iota(jnp.int32, s.shape, 1)
    idx = jnp.min(jnp.where(s <= m, ii, jnp.int32(_N)), axis=1, keepdims=True)
    out_ref[...] = jnp.concatenate([idx // _COLS, idx % _COLS], axis=1)


def kernel(x, weights):
    batch = x.shape[0]
    w = weights.reshape(_N, _D)
    return pl.pallas_call(
        _bmu_block,
        grid=(batch // _BB,),
        in_specs=[
            pl.BlockSpec((_BB, _D), lambda i: (i, 0)),
            pl.BlockSpec((_N, _D), lambda i: (0, 0)),
        ],
        out_specs=pl.BlockSpec((_BB, 2), lambda i: (i, 0)),
        out_shape=jax.ShapeDtypeStruct((batch, 2), jnp.int32),
    )(x, w)


# speedup vs baseline: 1.1691x; 1.1691x over previous
import jax
import jax.numpy as jnp
from jax.experimental import pallas as pl

_ROWS, _COLS, _D = 64, 64, 32
_N = _ROWS * _COLS
_BB = 1024


def _bmu_block(x_ref, wt_ref, out_ref):
    xb = x_ref[...]
    wt = wt_ref[...]
    wn2 = jnp.sum(wt * wt, axis=0, keepdims=True)
    x_aug = jnp.concatenate(
        [xb, jnp.ones((xb.shape[0], 1), jnp.float32)], axis=1)
    wt_aug = jnp.concatenate([wt * -2.0, wn2], axis=0)
    s = jnp.dot(
        x_aug, wt_aug,
        preferred_element_type=jnp.float32,
        precision=jax.lax.Precision.HIGHEST,
    )
    nt = _N // 128
    m_lane = s[:, 0:128]
    t_best = jnp.zeros(m_lane.shape, jnp.int32)
    for t in range(1, nt):
        cur = s[:, t * 128:(t + 1) * 128]
        mask = cur < m_lane
        m_lane = jnp.where(mask, cur, m_lane)
        t_best = jnp.where(mask, jnp.int32(t), t_best)
    m = jnp.min(m_lane, axis=1, keepdims=True)
    lane = jax.lax.broadcasted_iota(jnp.int32, m_lane.shape, 1)
    flat = t_best * 128 + lane
    idx = jnp.min(jnp.where(m_lane <= m, flat, jnp.int32(_N)),
                  axis=1, keepdims=True)
    out_ref[...] = jnp.concatenate([idx // _COLS, idx % _COLS], axis=1)


def kernel(x, weights):
    batch = x.shape[0]
    wt = weights.reshape(_N, _D).T
    return pl.pallas_call(
        _bmu_block,
        grid=(batch // _BB,),
        in_specs=[
            pl.BlockSpec((_BB, _D), lambda i: (i, 0)),
            pl.BlockSpec((_D, _N), lambda i: (0, 0)),
        ],
        out_specs=pl.BlockSpec((_BB, 2), lambda i: (i, 0)),
        out_shape=jax.ShapeDtypeStruct((batch, 2), jnp.int32),
    )(x, wt)
